# baseline (speedup 1.0000x reference)
import functools

import jax
import jax.numpy as jnp
from jax import lax
from jax.experimental import pallas as pl
from jax.experimental.pallas import tpu as pltpu
from jax.experimental.pallas import tpu_sc as plsc

VOCAB = 1000001
EMBED_DIM = 32
BATCH = 16384

_NC = 2
_NS = 16
_NW = _NC * _NS
_PHYS_MINOR = 1000064
_SCOLS = 1024
_NSTRIP = 977
_LAST_STRIP = _NSTRIP - 1
_LAST_OFF = _PHYS_MINOR - _SCOLS
_LAST_ADJ = _LAST_STRIP * _SCOLS - _LAST_OFF
_KSLOT = 64
_MAXE = 1024
_TRASH_ROWS = 512
_OUT_ROWS = BATCH + _TRASH_ROWS

_mesh = plsc.VectorSubcoreMesh(core_axis_name="c", subcore_axis_name="s")


@functools.partial(
    pl.kernel,
    mesh=_mesh,
    out_type=jax.ShapeDtypeStruct((_OUT_ROWS, 128), jnp.float32),
    scratch_types=[
        pltpu.VMEM((BATCH,), jnp.int32),
        pltpu.VMEM((_MAXE + 16,), jnp.int32),
        pltpu.VMEM((_MAXE + 16,), jnp.int32),
        pltpu.VMEM((31 * _KSLOT,), jnp.int32),
        pltpu.VMEM((31 * _KSLOT,), jnp.int32),
        pltpu.VMEM((32,), jnp.int32),
        pltpu.VMEM((EMBED_DIM, _SCOLS), jnp.float32),
        pltpu.VMEM((EMBED_DIM, _SCOLS), jnp.float32),
        pltpu.VMEM((_KSLOT, 128), jnp.float32),
        pltpu.VMEM((_KSLOT, 128), jnp.float32),
        pltpu.SemaphoreType.DMA,
        pltpu.SemaphoreType.DMA,
        pltpu.SemaphoreType.DMA,
        pltpu.SemaphoreType.DMA,
    ],
    compiler_params=pltpu.CompilerParams(needs_layout_passes=False),
)
def _vp_gather_kernel(idx_hbm, table_t_hbm, out_hbm,
                      idx_all, my_idx, my_b, meta_col, meta_b, meta_cnt,
                      bank0, bank1, stg0, stg1, sem0, sem1, sem2, sem3):
    wid = lax.axis_index("s") * _NC + lax.axis_index("c")
    s_lo = lax.shift_right_logical(_NSTRIP * wid + 31, 5)
    s_hi = lax.shift_right_logical(_NSTRIP * (wid + 1) + 31, 5)
    nstrips = s_hi - s_lo

    pltpu.sync_copy(idx_hbm, idx_all)

    iota16 = lax.iota(jnp.int32, 16)

    def fire(s_rel, bank_ref, sem):
        s = s_lo + s_rel
        off = jnp.where(s == _LAST_STRIP, jnp.int32(_LAST_OFF), s * _SCOLS)
        off = pl.multiple_of(off, 128)
        return pltpu.async_copy(
            table_t_hbm.at[:, pl.ds(off, _SCOLS)], bank_ref, sem)

    def drain(bank_ref, sem):
        pltpu.make_async_copy(
            table_t_hbm.at[:, pl.ds(0, _SCOLS)], bank_ref, sem).wait()

    fire(0, bank0, sem0)
    fire(1, bank1, sem1)

    def scan_body(k, pos):
        v = idx_all[pl.ds(k * 16, 16)]
        sv = lax.shift_right_logical(v, 10)
        m = jnp.logical_and(sv >= s_lo, sv < s_hi)
        plsc.store_compressed(my_idx.at[pl.ds(pos, 16)], v, mask=m)
        plsc.store_compressed(my_b.at[pl.ds(pos, 16)], iota16 + k * 16, mask=m)
        return pos + plsc.all_reduce_population_count(m)[0]

    n = lax.fori_loop(0, BATCH // 16, scan_body, jnp.int32(0))

    meta_cnt[pl.ds(0, 16)] = jnp.zeros((16,), jnp.int32)
    meta_cnt[pl.ds(16, 16)] = jnp.zeros((16,), jnp.int32)

    def bucket_body(j, carry):
        v16 = my_idx[pl.ds(j * 16, 16)]
        b16 = my_b[pl.ds(j * 16, 16)]
        for i in range(16):
            @pl.when(j * 16 + i < n)
            def _():
                c = v16[i]
                b = b16[i]
                sg = lax.shift_right_logical(c, 10)
                w_rel = sg - s_lo
                col = lax.bitwise_and(c, jnp.int32(_SCOLS - 1))
                col = col + jnp.where(sg == _LAST_STRIP,
                                      jnp.int32(_LAST_ADJ), jnp.int32(0))
                cnt = plsc.load_gather(
                    meta_cnt, [jnp.full((16,), w_rel, jnp.int32)])[0]
                slot = lax.min(cnt, jnp.int32(_KSLOT - 1))
                mi = jnp.full((16,), w_rel * _KSLOT + slot, jnp.int32)
                plsc.store_scatter(meta_col, [mi],
                                   jnp.full((16,), col, jnp.int32))
                plsc.store_scatter(meta_b, [mi],
                                   jnp.full((16,), b, jnp.int32))
                plsc.store_scatter(meta_cnt,
                                   [jnp.full((16,), w_rel, jnp.int32)],
                                   jnp.full((16,), cnt + 1, jnp.int32))
        return carry

    lax.fori_loop(0, (_MAXE + 15) // 16, bucket_body, jnp.int32(0))

    def extract(w_rel, bank_ref, stg, sem):
        cnt = plsc.load_gather(
            meta_cnt, [jnp.full((16,), w_rel, jnp.int32)])[0]
        for q in range(_KSLOT // 16):
            slots = q * 16 + iota16
            midx = jnp.full((16,), w_rel * _KSLOT, jnp.int32) + slots
            col_v = plsc.load_gather(meta_col, [midx])
            col_v = lax.bitwise_and(col_v, jnp.int32(_SCOLS - 1))
            b_v = plsc.load_gather(meta_b, [midx])
            valid = slots < cnt
            trash = BATCH + lax.bitwise_and(
                wid * 4 + q + w_rel * 13, jnp.int32(_TRASH_ROWS - 1))
            b_v = jnp.where(valid, b_v, trash)
            for r in range(EMBED_DIM):
                vals = plsc.load_gather(
                    bank_ref, [jnp.full((16,), r, jnp.int32), col_v])
                plsc.store_scatter(
                    stg, [slots, jnp.full((16,), r, jnp.int32)], vals)
            pltpu.async_copy(stg.at[pl.ds(q * 16, 16)], out_hbm.at[b_v], sem)

    def drain_sc(stg, sem):
        for q in range(_KSLOT // 16):
            pltpu.make_async_copy(
                stg.at[pl.ds(q * 16, 16)], out_hbm.at[iota16], sem).wait()

    for stg, sem in ((stg0, sem2), (stg1, sem3)):
        for q in range(_KSLOT // 16):
            pltpu.async_copy(stg.at[pl.ds(q * 16, 16)],
                             out_hbm.at[BATCH + iota16], sem)

    def body(t, carry):
        drain(bank0, sem0)
        drain_sc(stg0, sem2)
        extract(2 * t, bank0, stg0, sem2)
        fire(lax.rem(2 * t + 2, nstrips), bank0, sem0)
        drain(bank1, sem1)
        drain_sc(stg1, sem3)
        extract(2 * t + 1, bank1, stg1, sem3)
        fire(lax.rem(2 * t + 3, nstrips), bank1, sem1)
        return carry

    lax.fori_loop(0, 15, body, jnp.int32(0))
    drain(bank0, sem0)
    drain(bank1, sem1)
    drain_sc(stg0, sem2)
    extract(30, bank0, stg0, sem2)
    drain_sc(stg0, sem2)
    drain_sc(stg1, sem3)


def kernel(inputs, table):
    idx = inputs.astype(jnp.int32)
    out_pad = _vp_gather_kernel(idx, table.T)
    return out_pad[:BATCH, :EMBED_DIM]


# speedup vs baseline: 1.3142x; 1.3142x over previous
import functools

import jax
import jax.numpy as jnp
from jax import lax
from jax.experimental import pallas as pl
from jax.experimental.pallas import tpu as pltpu
from jax.experimental.pallas import tpu_sc as plsc

VOCAB = 1000001
EMBED_DIM = 32
BATCH = 16384

_NC = 2
_NS = 16
_NW = _NC * _NS
_B_PER_W = BATCH // _NW
_G = 8
_NGROUP = _B_PER_W // _G

_mesh = plsc.VectorSubcoreMesh(core_axis_name="c", subcore_axis_name="s")


@functools.partial(
    pl.kernel,
    mesh=_mesh,
    out_type=jax.ShapeDtypeStruct((EMBED_DIM, BATCH), jnp.float32),
    scratch_types=[
        pltpu.VMEM((_B_PER_W + 16,), jnp.int32),
    ] + [
        pltpu.VMEM((EMBED_DIM, 128), jnp.float32)
        for _ in range(2 * _G)
    ] + [
        pltpu.VMEM((EMBED_DIM, _B_PER_W), jnp.float32),
        pltpu.SemaphoreType.DMA,
        pltpu.SemaphoreType.DMA,
    ],
    compiler_params=pltpu.CompilerParams(needs_layout_passes=False),
)
def _colgather_kernel(idx_hbm, table_t_hbm, out_t_hbm,
                      idx_v, *rest):
    slots = [list(rest[0:_G]), list(rest[_G:2 * _G])]
    out_block, sem0, sem1 = rest[2 * _G], rest[2 * _G + 1], rest[2 * _G + 2]
    wid = lax.axis_index("s") * _NC + lax.axis_index("c")
    base = wid * _B_PER_W
    pltpu.sync_copy(idx_hbm.at[pl.ds(base, _B_PER_W)], idx_v.at[pl.ds(0, _B_PER_W)])
    idx_v[pl.ds(_B_PER_W, 16)] = jnp.zeros((16,), jnp.int32)

    lanes16 = lax.iota(jnp.int32, 16)

    def fire(g, bank, sem):
        gb = g * _G
        cvec = idx_v[pl.ds(gb, 16)]
        for i in range(_G):
            c = cvec[i]
            off = lax.shift_left(lax.shift_right_logical(c, 7), 7)
            off = pl.multiple_of(off, 128)
            pltpu.async_copy(
                table_t_hbm.at[:, pl.ds(off, 128)],
                slots[bank][i],
                sem,
            )

    def drain(bank, sem):
        for i in range(_G):
            pltpu.make_async_copy(
                table_t_hbm.at[:, pl.ds(0, 128)],
                slots[bank][i],
                sem,
            ).wait()

    def extract(g, bank):
        gb = g * _G
        for i in range(_G):
            c_b = plsc.load_gather(idx_v, [jnp.full((16,), gb + i, jnp.int32)])
            lane_b = lax.bitwise_and(c_b, jnp.int32(127))
            col_b = jnp.full((16,), gb + i, jnp.int32)
            lo = plsc.load_gather(slots[bank][i], [lanes16, lane_b])
            hi = plsc.load_gather(slots[bank][i], [lanes16 + 16, lane_b])
            plsc.store_scatter(out_block, [lanes16, col_b], lo)
            plsc.store_scatter(out_block, [lanes16 + 16, col_b], hi)

    fire(0, 0, sem0)

    def body(j, carry):
        g0 = 2 * j
        fire(g0 + 1, 1, sem1)
        drain(0, sem0)
        extract(g0, 0)
        g_next = lax.rem(g0 + 2, _NGROUP)
        fire(g_next, 0, sem0)
        drain(1, sem1)
        extract(g0 + 1, 1)
        return carry

    lax.fori_loop(0, _NGROUP // 2, body, 0)
    drain(0, sem0)

    pltpu.sync_copy(out_block, out_t_hbm.at[:, pl.ds(base, _B_PER_W)])


def kernel(inputs, table):
    idx = inputs.astype(jnp.int32)
    out_t = _colgather_kernel(idx, table.T)
    return out_t.T
